# speedup vs baseline: 1.1505x; 1.1505x over previous
import functools

import jax
import jax.numpy as jnp
from jax import lax
from jax.experimental import pallas as pl
from jax.experimental.pallas import tpu as pltpu
from jax.experimental.pallas import tpu_sc as plsc


def _sc_gather(emb, xflat, *, chunk=128):
    n_total, emb_dim = xflat.shape[0], emb.shape[1]
    info = plsc.get_sparse_core_info()
    nc, ns = info.num_cores, info.num_subcores
    nw = nc * ns
    n_per_w = n_total // nw
    assert n_per_w * nw == n_total and n_per_w % chunk == 0
    n_iters = n_per_w // chunk

    mesh = plsc.VectorSubcoreMesh(core_axis_name="c", subcore_axis_name="s")

    @functools.partial(
        pl.kernel,
        mesh=mesh,
        out_type=jax.ShapeDtypeStruct((n_total, emb_dim), jnp.float32),
        scratch_types=[
            pltpu.VMEM((chunk,), jnp.int32),
            pltpu.VMEM((chunk, emb_dim), jnp.float32),
            pltpu.SemaphoreType.DMA,
        ],
    )
    def gather_kernel(emb_hbm, idx_hbm, out_hbm, idx_v, rows_v, sem):
        wid = lax.axis_index("s") * nc + lax.axis_index("c")
        base = wid * n_per_w

        def body(i, carry):
            off = base + i * chunk
            pltpu.sync_copy(idx_hbm.at[pl.ds(off, chunk)], idx_v)
            pltpu.async_copy(emb_hbm.at[idx_v], rows_v, sem).wait()
            pltpu.sync_copy(rows_v, out_hbm.at[pl.ds(off, chunk)])
            return carry

        lax.fori_loop(0, n_iters, body, 0)

    return gather_kernel(emb, xflat)


def _fc_in_kernel(g_ref, w_ref, b_ref, o_ref):
    k = pl.program_id(0)

    @pl.when(k == 0)
    def _():
        o_ref[...] = jnp.broadcast_to(b_ref[...], o_ref.shape)

    o_ref[...] += lax.dot_general(
        g_ref[...], w_ref[...], (((1,), (1,)), ((), ())),
        preferred_element_type=jnp.float32)


def _fc_in(g, w_in, b_in, *, k_tile=3200):
    b, k_total = g.shape
    v = w_in.shape[0]
    n_k = k_total // k_tile
    assert n_k * k_tile == k_total
    return pl.pallas_call(
        _fc_in_kernel,
        grid=(n_k,),
        in_specs=[
            pl.BlockSpec((b, k_tile), lambda k: (0, k)),
            pl.BlockSpec((v, k_tile), lambda k: (0, k)),
            pl.BlockSpec((1, v), lambda k: (0, 0)),
        ],
        out_specs=pl.BlockSpec((b, v), lambda k: (0, 0)),
        out_shape=jax.ShapeDtypeStruct((b, v), jnp.float32),
    )(g, w_in, b_in)


def _fc_out_kernel(h_ref, w_ref, b_ref, o_ref):
    o_ref[...] = lax.dot_general(
        h_ref[...], w_ref[...], (((1,), (1,)), ((), ())),
        preferred_element_type=jnp.float32) + b_ref[...]


def _fc_out(h, w_out, b_out, *, v_tile=2048):
    b, v = h.shape
    vocab = w_out.shape[0]
    n_v = pl.cdiv(vocab, v_tile)
    return pl.pallas_call(
        _fc_out_kernel,
        grid=(n_v,),
        in_specs=[
            pl.BlockSpec((b, v), lambda j: (0, 0)),
            pl.BlockSpec((v_tile, v), lambda j: (j, 0)),
            pl.BlockSpec((1, v_tile), lambda j: (0, j)),
        ],
        out_specs=pl.BlockSpec((b, v_tile), lambda j: (0, j)),
        out_shape=jax.ShapeDtypeStruct((b, vocab), jnp.float32),
    )(h, w_out, b_out)


def kernel(x, emb, W_in, b_in, W_out, b_out):
    b, ctx = x.shape
    emb_dim = emb.shape[1]
    xflat = x.reshape(-1)
    g = _sc_gather(emb, xflat)
    g = g.reshape(b, ctx * emb_dim)
    h = _fc_in(g, W_in, b_in.reshape(1, -1))
    return _fc_out(h, W_out, b_out.reshape(1, -1))


# speedup vs baseline: 1.2451x; 1.0822x over previous
import functools

import jax
import jax.numpy as jnp
from jax import lax
from jax.experimental import pallas as pl
from jax.experimental.pallas import tpu as pltpu
from jax.experimental.pallas import tpu_sc as plsc


def _sc_gather(emb, xflat, *, chunk=128):
    n_total, emb_dim = xflat.shape[0], emb.shape[1]
    info = plsc.get_sparse_core_info()
    nc, ns = info.num_cores, info.num_subcores
    nw = nc * ns
    n_per_w = n_total // nw
    assert n_per_w * nw == n_total and n_per_w % chunk == 0
    n_iters = n_per_w // chunk

    mesh = plsc.VectorSubcoreMesh(core_axis_name="c", subcore_axis_name="s")

    @functools.partial(
        pl.kernel,
        mesh=mesh,
        out_type=jax.ShapeDtypeStruct((n_total, emb_dim), jnp.float32),
        scratch_types=[
            pltpu.VMEM((chunk,), jnp.int32),
            pltpu.VMEM((chunk, emb_dim), jnp.float32),
            pltpu.SemaphoreType.DMA,
        ],
    )
    def gather_kernel(emb_hbm, idx_hbm, out_hbm, idx_v, rows_v, sem):
        wid = lax.axis_index("s") * nc + lax.axis_index("c")
        base = wid * n_per_w

        def body(i, carry):
            off = base + i * chunk
            pltpu.sync_copy(idx_hbm.at[pl.ds(off, chunk)], idx_v)
            pltpu.async_copy(emb_hbm.at[idx_v], rows_v, sem).wait()
            pltpu.sync_copy(rows_v, out_hbm.at[pl.ds(off, chunk)])
            return carry

        lax.fori_loop(0, n_iters, body, 0)

    return gather_kernel(emb, xflat)


def _fc_in_kernel(g_ref, w_ref, b_ref, o_ref, *, c_tile):
    k = pl.program_id(0)

    @pl.when(k == 0)
    def _():
        o_ref[...] = jnp.broadcast_to(b_ref[...], o_ref.shape)

    acc = o_ref[...]
    for c in range(c_tile):
        acc += lax.dot_general(
            g_ref[:, c, :], w_ref[:, c, :], (((1,), (1,)), ((), ())),
            preferred_element_type=jnp.float32)
    o_ref[...] = acc


def _fc_in(g, w_in, b_in, *, c_tile=8):
    b, ctx, emb_dim = g.shape
    v = w_in.shape[0]
    w3 = w_in.reshape(v, ctx, emb_dim)
    n_c = ctx // c_tile
    assert n_c * c_tile == ctx
    return pl.pallas_call(
        functools.partial(_fc_in_kernel, c_tile=c_tile),
        grid=(n_c,),
        in_specs=[
            pl.BlockSpec((b, c_tile, emb_dim), lambda k: (0, k, 0)),
            pl.BlockSpec((v, c_tile, emb_dim), lambda k: (0, k, 0)),
            pl.BlockSpec((1, v), lambda k: (0, 0)),
        ],
        out_specs=pl.BlockSpec((b, v), lambda k: (0, 0)),
        out_shape=jax.ShapeDtypeStruct((b, v), jnp.float32),
    )(g, w3, b_in)


def _fc_out_kernel(h_ref, w_ref, b_ref, o_ref):
    o_ref[...] = lax.dot_general(
        h_ref[...], w_ref[...], (((1,), (1,)), ((), ())),
        preferred_element_type=jnp.float32) + b_ref[...]


def _fc_out(h, w_out, b_out, *, v_tile=2048):
    b, v = h.shape
    vocab = w_out.shape[0]
    n_v = pl.cdiv(vocab, v_tile)
    return pl.pallas_call(
        _fc_out_kernel,
        grid=(n_v,),
        in_specs=[
            pl.BlockSpec((b, v), lambda j: (0, 0)),
            pl.BlockSpec((v_tile, v), lambda j: (j, 0)),
            pl.BlockSpec((1, v_tile), lambda j: (0, j)),
        ],
        out_specs=pl.BlockSpec((b, v_tile), lambda j: (0, j)),
        out_shape=jax.ShapeDtypeStruct((b, vocab), jnp.float32),
    )(h, w_out, b_out)


def kernel(x, emb, W_in, b_in, W_out, b_out):
    b, ctx = x.shape
    emb_dim = emb.shape[1]
    xflat = x.reshape(-1)
    g = _sc_gather(emb, xflat)
    g = g.reshape(b, ctx, emb_dim)
    h = _fc_in(g, W_in, b_in.reshape(1, -1))
    return _fc_out(h, W_out, b_out.reshape(1, -1))


# speedup vs baseline: 1.2512x; 1.0050x over previous
import functools

import jax
import jax.numpy as jnp
from jax import lax
from jax.experimental import pallas as pl
from jax.experimental.pallas import tpu as pltpu
from jax.experimental.pallas import tpu_sc as plsc


def _sc_gather(emb, xflat, *, chunk=128):
    n_total, emb_dim = xflat.shape[0], emb.shape[1]
    info = plsc.get_sparse_core_info()
    nc, ns = info.num_cores, info.num_subcores
    nw = nc * ns
    n_per_w = n_total // nw
    assert n_per_w * nw == n_total and n_per_w % chunk == 0
    n_iters = n_per_w // chunk

    mesh = plsc.VectorSubcoreMesh(core_axis_name="c", subcore_axis_name="s")

    @functools.partial(
        pl.kernel,
        mesh=mesh,
        out_type=jax.ShapeDtypeStruct((n_total, emb_dim), jnp.float32),
        scratch_types=[
            pltpu.VMEM((chunk,), jnp.int32),
            pltpu.VMEM((chunk, emb_dim), jnp.float32),
            pltpu.SemaphoreType.DMA,
        ],
    )
    def gather_kernel(emb_hbm, idx_hbm, out_hbm, idx_v, rows_v, sem):
        wid = lax.axis_index("s") * nc + lax.axis_index("c")
        base = wid * n_per_w

        def body(i, carry):
            off = base + i * chunk
            pltpu.sync_copy(idx_hbm.at[pl.ds(off, chunk)], idx_v)
            pltpu.async_copy(emb_hbm.at[idx_v], rows_v, sem).wait()
            pltpu.sync_copy(rows_v, out_hbm.at[pl.ds(off, chunk)])
            return carry

        lax.fori_loop(0, n_iters, body, 0)

    return gather_kernel(emb, xflat)


def _fc_in_kernel(g_ref, w_ref, b_ref, o_ref, *, c_tile):
    k = pl.program_id(0)

    @pl.when(k == 0)
    def _():
        o_ref[...] = jnp.broadcast_to(b_ref[...], o_ref.shape)

    acc = o_ref[...]
    for c in range(c_tile):
        acc += lax.dot_general(
            g_ref[:, c, :], w_ref[:, c, :], (((1,), (1,)), ((), ())),
            preferred_element_type=jnp.float32)
    o_ref[...] = acc


def _fc_in(g, w_in, b_in, *, c_tile=8):
    b, ctx, emb_dim = g.shape
    v = w_in.shape[0]
    w3 = w_in.reshape(v, ctx, emb_dim)
    n_c = ctx // c_tile
    assert n_c * c_tile == ctx
    return pl.pallas_call(
        functools.partial(_fc_in_kernel, c_tile=c_tile),
        grid=(n_c,),
        in_specs=[
            pl.BlockSpec((b, c_tile, emb_dim), lambda k: (0, k, 0)),
            pl.BlockSpec((v, c_tile, emb_dim), lambda k: (0, k, 0)),
            pl.BlockSpec((1, v), lambda k: (0, 0)),
        ],
        out_specs=pl.BlockSpec((b, v), lambda k: (0, 0)),
        out_shape=jax.ShapeDtypeStruct((b, v), jnp.float32),
    )(g, w3, b_in)


def _fc_out_kernel(h_ref, w_ref, b_ref, o_ref):
    o_ref[...] = lax.dot_general(
        h_ref[...], w_ref[...], (((1,), (1,)), ((), ())),
        preferred_element_type=jnp.float32) + b_ref[...]


def _fc_out(h, w_out, b_out, *, v_tile=4096):
    b, v = h.shape
    vocab = w_out.shape[0]
    n_v = pl.cdiv(vocab, v_tile)
    return pl.pallas_call(
        _fc_out_kernel,
        grid=(n_v,),
        in_specs=[
            pl.BlockSpec((b, v), lambda j: (0, 0)),
            pl.BlockSpec((v_tile, v), lambda j: (j, 0)),
            pl.BlockSpec((1, v_tile), lambda j: (0, j)),
        ],
        out_specs=pl.BlockSpec((b, v_tile), lambda j: (0, j)),
        out_shape=jax.ShapeDtypeStruct((b, vocab), jnp.float32),
    )(h, w_out, b_out)


def kernel(x, emb, W_in, b_in, W_out, b_out):
    b, ctx = x.shape
    emb_dim = emb.shape[1]
    xflat = x.reshape(-1)
    g = _sc_gather(emb, xflat)
    g = g.reshape(b, ctx, emb_dim)
    h = _fc_in(g, W_in, b_in.reshape(1, -1))
    return _fc_out(h, W_out, b_out.reshape(1, -1))


# speedup vs baseline: 1.2952x; 1.0352x over previous
import functools

import jax
import jax.numpy as jnp
from jax import lax
from jax.experimental import pallas as pl
from jax.experimental.pallas import tpu as pltpu
from jax.experimental.pallas import tpu_sc as plsc


def _sc_gather(emb, xflat, *, chunk=128):
    n_total, emb_dim = xflat.shape[0], emb.shape[1]
    info = plsc.get_sparse_core_info()
    nc, ns = info.num_cores, info.num_subcores
    nw = nc * ns
    n_per_w = n_total // nw
    assert n_per_w * nw == n_total and n_per_w % chunk == 0
    n_iters = n_per_w // chunk

    mesh = plsc.VectorSubcoreMesh(core_axis_name="c", subcore_axis_name="s")

    assert n_iters % 2 == 0

    @functools.partial(
        pl.kernel,
        mesh=mesh,
        out_type=jax.ShapeDtypeStruct((n_total, emb_dim), jnp.float32),
        scratch_types=[
            pltpu.VMEM((chunk,), jnp.int32),
            pltpu.VMEM((chunk,), jnp.int32),
            pltpu.VMEM((chunk, emb_dim), jnp.float32),
            pltpu.VMEM((chunk, emb_dim), jnp.float32),
            pltpu.SemaphoreType.DMA,
            pltpu.SemaphoreType.DMA,
        ],
    )
    def gather_kernel(emb_hbm, idx_hbm, out_hbm, idx_a, idx_b, rows_a,
                      rows_b, sem_g, sem_o):
        wid = lax.axis_index("s") * nc + lax.axis_index("c")
        base = wid * n_per_w

        def phase(i, idx_cur, rows_cur, idx_nxt, rows_nxt):
            off = base + i * chunk
            pltpu.make_async_copy(emb_hbm.at[idx_cur], rows_cur, sem_g).wait()

            @pl.when(i > 0)
            def _():
                pltpu.make_async_copy(
                    rows_nxt, out_hbm.at[pl.ds(off, chunk)], sem_o).wait()

            pltpu.async_copy(rows_cur, out_hbm.at[pl.ds(off, chunk)], sem_o)

            @pl.when(i + 1 < n_iters)
            def _():
                pltpu.sync_copy(
                    idx_hbm.at[pl.ds(off + chunk, chunk)], idx_nxt)
                pltpu.async_copy(emb_hbm.at[idx_nxt], rows_nxt, sem_g)

        pltpu.sync_copy(idx_hbm.at[pl.ds(base, chunk)], idx_a)
        pltpu.async_copy(emb_hbm.at[idx_a], rows_a, sem_g)

        def body(j, carry):
            phase(2 * j, idx_a, rows_a, idx_b, rows_b)
            phase(2 * j + 1, idx_b, rows_b, idx_a, rows_a)
            return carry

        lax.fori_loop(0, n_iters // 2, body, 0)
        pltpu.make_async_copy(
            rows_b, out_hbm.at[pl.ds(base, chunk)], sem_o).wait()

    return gather_kernel(emb, xflat)


def _fc_in_kernel(g_ref, w_ref, b_ref, o_ref, *, c_tile):
    k = pl.program_id(0)

    @pl.when(k == 0)
    def _():
        o_ref[...] = jnp.broadcast_to(b_ref[...], o_ref.shape)

    acc = o_ref[...]
    for c in range(c_tile):
        acc += lax.dot_general(
            g_ref[:, c, :], w_ref[:, c, :], (((1,), (1,)), ((), ())),
            preferred_element_type=jnp.float32)
    o_ref[...] = acc


def _fc_in(g, w_in, b_in, *, c_tile=8):
    b, ctx, emb_dim = g.shape
    v = w_in.shape[0]
    w3 = w_in.reshape(v, ctx, emb_dim)
    n_c = ctx // c_tile
    assert n_c * c_tile == ctx
    return pl.pallas_call(
        functools.partial(_fc_in_kernel, c_tile=c_tile),
        grid=(n_c,),
        in_specs=[
            pl.BlockSpec((b, c_tile, emb_dim), lambda k: (0, k, 0)),
            pl.BlockSpec((v, c_tile, emb_dim), lambda k: (0, k, 0)),
            pl.BlockSpec((1, v), lambda k: (0, 0)),
        ],
        out_specs=pl.BlockSpec((b, v), lambda k: (0, 0)),
        out_shape=jax.ShapeDtypeStruct((b, v), jnp.float32),
    )(g, w3, b_in)


def _fc_out_kernel(h_ref, w_ref, b_ref, o_ref):
    o_ref[...] = lax.dot_general(
        h_ref[...], w_ref[...], (((1,), (1,)), ((), ())),
        preferred_element_type=jnp.float32) + b_ref[...]


def _fc_out(h, w_out, b_out, *, v_tile=4096):
    b, v = h.shape
    vocab = w_out.shape[0]
    n_v = pl.cdiv(vocab, v_tile)
    return pl.pallas_call(
        _fc_out_kernel,
        grid=(n_v,),
        in_specs=[
            pl.BlockSpec((b, v), lambda j: (0, 0)),
            pl.BlockSpec((v_tile, v), lambda j: (j, 0)),
            pl.BlockSpec((1, v_tile), lambda j: (0, j)),
        ],
        out_specs=pl.BlockSpec((b, v_tile), lambda j: (0, j)),
        out_shape=jax.ShapeDtypeStruct((b, vocab), jnp.float32),
    )(h, w_out, b_out)


def kernel(x, emb, W_in, b_in, W_out, b_out):
    b, ctx = x.shape
    emb_dim = emb.shape[1]
    xflat = x.reshape(-1)
    g = _sc_gather(emb, xflat)
    g = g.reshape(b, ctx, emb_dim)
    h = _fc_in(g, W_in, b_in.reshape(1, -1))
    return _fc_out(h, W_out, b_out.reshape(1, -1))
